# Initial kernel scaffold; baseline (speedup 1.0000x reference)
import jax
import jax.numpy as jnp
from jax.experimental import pallas as pl


def kernel(x, edge_index, batch, params):
    raise NotImplementedError("write your pallas kernel here")


# speedup vs baseline: 1.1281x; 1.1281x over previous
import jax
import jax.numpy as jnp
from jax.experimental import pallas as pl

N, E, D, G, HEADS = 10000, 320000, 128, 64, 8


def _gat_softmax_coef(e, dst):
    ee = jnp.exp(e)
    denom = jax.ops.segment_sum(ee, dst, num_segments=N)
    return ee / (denom[dst] + 1e-16), ee


def _gat_hop(h, att_s, att_d, src, dst):
    a_s = (h * att_s[None]).sum(-1)
    a_d = (h * att_d[None]).sum(-1)
    e = jax.nn.leaky_relu(a_s[src] + a_d[dst], 0.2)
    coef, _ = _gat_softmax_coef(e, dst)
    return jax.ops.segment_sum(h[src] * coef[:, :, None], dst, num_segments=N)


def _gat2_tail(g, p, src, dst, hops):
    h = (g @ p['gat2_W']).reshape(N, 1, D)
    for _ in range(hops):
        h = _gat_hop(h, p['gat2_as'], p['gat2_ad'], src, dst)
    h = h.reshape(N, D) + p['gat2_b']
    h = jax.nn.relu(h)
    h = jax.nn.relu(h @ p['gatA_W'] + p['gatA_b'])
    h = jax.nn.relu(h @ p['gatB_W'] + p['gatB_b'])
    return h @ p['gatC_W'] + p['gatC_b']


def _gcn_layer(h_in, src, dst, W, b, norm, inv_deg, hops):
    h = h_in @ W
    for _ in range(hops):
        h = jax.ops.segment_sum(h[src] * norm[:, None], dst, num_segments=N) \
            + h * inv_deg[:, None]
    return h + b


def kernel(x, edge_index, batch, params):
    p = params
    src = edge_index[0]
    dst = edge_index[1]

    h = (x @ p['gat1_W']).reshape(N, HEADS, D)
    g = []
    for _ in range(3):
        h = _gat_hop(h, p['gat1_as'], p['gat1_ad'], src, dst)
        g.append(jax.nn.elu(h.reshape(N, HEADS * D) + p['gat1_b']))

    b1 = jax.nn.relu(_gat2_tail(g[0], p, src, dst, 1))
    b2 = jax.nn.relu(_gat2_tail(g[1], p, src, dst, 2))
    h3 = jax.nn.relu(_gat2_tail(g[2], p, src, dst, 3))

    deg = jnp.ones((N,), jnp.float32).at[dst].add(1.0)
    dinv = deg ** -0.5
    norm = dinv[src] * dinv[dst]
    inv_deg = dinv * dinv

    h1 = jax.nn.relu(_gcn_layer(b1, src, dst, p['gcn2_W'], p['gcn2_b'], norm, inv_deg, 1))
    h1 = jax.nn.relu(_gcn_layer(h1, src, dst, p['gcn3_W'], p['gcn3_b'], norm, inv_deg, 1))
    h2 = jax.nn.relu(_gcn_layer(b2, src, dst, p['gcn2_W'], p['gcn2_b'], norm, inv_deg, 2))

    a = h1 @ p['hwA_W'] + p['hwA_b']
    b = h2 @ p['hwB_W'] + p['hwB_b']
    z = jax.nn.sigmoid(a + b)
    hmix = z * b + (1.0 - z) * a
    gi = h3 @ p['gru_Wi'] + p['gru_bi']
    gh = hmix @ p['gru_Wh'] + p['gru_bh']
    i_r, i_z, i_n = jnp.split(gi, 3, axis=-1)
    h_r, h_z, h_n = jnp.split(gh, 3, axis=-1)
    r = jax.nn.sigmoid(i_r + h_r)
    zz = jax.nn.sigmoid(i_z + h_z)
    nn_ = jnp.tanh(i_n + r * h_n)
    concat = (1.0 - zz) * nn_ + zz * hmix

    vi = jax.ops.segment_max(concat, batch, num_segments=G)
    vi = jnp.where(jnp.isfinite(vi), vi, 0.0)
    vi = jax.nn.relu(vi @ p['fA_W'] + p['fA_b'])
    return vi @ p['fB_W'] + p['fB_b']


# speedup vs baseline: 9.2853x; 8.2309x over previous
import functools

import jax
import jax.numpy as jnp
from jax import lax
from jax.experimental import pallas as pl
from jax.experimental.pallas import tpu as pltpu
from jax.experimental.pallas import tpu_sc as plsc

N, E, D, G, HEADS = 10000, 320000, 128, 64, 8
NP = 10240

NC = 2
NS = 16
EPS = E // NS
EPW = E // (NS * NC)
C = 80
RPT = NP // NS

_mesh_cache = []


def _mesh():
    if not _mesh_cache:
        _mesh_cache.append(plsc.VectorSubcoreMesh(
            core_axis_name="c", subcore_axis_name="s"))
    return _mesh_cache[0]


def _zero_fill(buf, n16):
    z = jnp.zeros((16,), jnp.float32)

    def body(i, _):
        buf[pl.ds(i * 16, 16)] = z
        return 0

    lax.fori_loop(0, n16, body, 0)


def _zero_fill2d(buf, nrows, ncols):
    z = jnp.zeros((16,), jnp.float32)

    def body(r, _):
        for k in range(ncols // 16):
            buf[r, pl.ds(k * 16, 16)] = z
        return 0

    lax.fori_loop(0, nrows, body, 0)


@functools.partial(jax.jit, static_argnames=("H",))
def _edge_scalars(asd, src, dst, *, H):
    HPC = (H + 1) // 2

    @functools.partial(
        pl.kernel, mesh=_mesh(),
        compiler_params=pltpu.CompilerParams(use_tc_tiling_on_sc=False, needs_layout_passes=False),
        out_type=(jax.ShapeDtypeStruct((H, E), jnp.float32),
                  jax.ShapeDtypeStruct((H, NP), jnp.float32)),
        scratch_types=[
            pltpu.VMEM((HPC, 2, NP), jnp.float32),
            pltpu.VMEM((C,), jnp.int32),
            pltpu.VMEM((C,), jnp.int32),
            pltpu.VMEM((HPC, C), jnp.float32),
            pltpu.VMEM((RPT,), jnp.float32),
            pltpu.VMEM_SHARED((HPC, NP), jnp.float32),
            pltpu.SemaphoreType.DMA,
        ])
    def kern(asd_h, src_h, dst_h, ee_h, den_h, tabs, sbuf, dbuf, eebuf, zbuf,
             dsh, sem):
        c = lax.axis_index("c")
        s = lax.axis_index("s")
        active = c * HPC < H

        _zero_fill(zbuf, RPT // 16)
        for hh in range(HPC):
            pltpu.sync_copy(zbuf, dsh.at[hh, pl.ds(s * RPT, RPT)])
        plsc.subcore_barrier()

        @pl.when(active)
        def _():
            for hh in range(HPC):
                pltpu.sync_copy(asd_h.at[0, c * HPC + hh], tabs.at[hh, 0])
                pltpu.sync_copy(asd_h.at[1, c * HPC + hh], tabs.at[hh, 1])

            def chunk(i, _):
                base = s * EPS + i * C
                pltpu.sync_copy(src_h.at[pl.ds(base, C)], sbuf)
                pltpu.sync_copy(dst_h.at[pl.ds(base, C)], dbuf)
                for hh in range(HPC):
                    for j in range(C // 16):
                        s16 = sbuf[pl.ds(j * 16, 16)]
                        d16 = dbuf[pl.ds(j * 16, 16)]
                        av = plsc.load_gather(tabs.at[hh, 0], [s16])
                        bv = plsc.load_gather(tabs.at[hh, 1], [d16])
                        e16 = av + bv
                        e16 = jnp.where(e16 >= 0.0, e16, e16 * 0.2)
                        eebuf[hh, pl.ds(j * 16, 16)] = jnp.exp(e16)
                for hh in range(HPC):
                    pltpu.sync_copy(eebuf.at[hh],
                                    ee_h.at[c * HPC + hh, pl.ds(base, C)])
                    pltpu.sync_copy(eebuf.at[hh], dsh.at[hh].at[dbuf],
                                    add=True)
                return 0

            lax.fori_loop(0, EPS // C, chunk, 0)

        plsc.subcore_barrier()

        @pl.when(active)
        def _():
            for hh in range(HPC):
                pltpu.sync_copy(dsh.at[hh, pl.ds(s * RPT, RPT)],
                                den_h.at[c * HPC + hh, pl.ds(s * RPT, RPT)])

    return kern(asd, src, dst)


@functools.partial(jax.jit, static_argnames=("slots", "weighted"))
def _spmm(table, src, dst, ee=None, *, slots, weighted):
    @functools.partial(
        pl.kernel, mesh=_mesh(),
        compiler_params=pltpu.CompilerParams(use_tc_tiling_on_sc=False, needs_layout_passes=False),
        out_type=jax.ShapeDtypeStruct((NC, NP, D), jnp.float32),
        scratch_types=[
            pltpu.VMEM((C,), jnp.int32),
            pltpu.VMEM((C,), jnp.int32),
            pltpu.VMEM((C, D), jnp.float32),
            pltpu.VMEM((C,), jnp.float32),
            pltpu.VMEM((C, D), jnp.float32),
            pltpu.VMEM_SHARED((NP, D), jnp.float32),
            pltpu.SemaphoreType.DMA,
        ])
    def kern(tbl_h, src_h, dst_h, ee_h, out_h, sbuf, dbuf, rows, wbuf, zrows,
             acc, sem):
        c = lax.axis_index("c")
        s = lax.axis_index("s")

        _zero_fill2d(zrows, C, D)
        for r in range(RPT // C):
            pltpu.sync_copy(zrows, acc.at[pl.ds(s * RPT + r * C, C)])
        plsc.subcore_barrier()

        nchunks = (EPS if slots == 2 else EPW) // C

        def chunk(i, _):
            if slots == 2:
                base = s * EPS + i * C
            else:
                base = (s * NC + c) * EPW + i * C
            pltpu.sync_copy(src_h.at[pl.ds(base, C)], sbuf)
            pltpu.sync_copy(dst_h.at[pl.ds(base, C)], dbuf)
            if slots == 2:
                pltpu.async_copy(tbl_h.at[c].at[sbuf], rows, sem).wait()
            else:
                pltpu.async_copy(tbl_h.at[0].at[sbuf], rows, sem).wait()
            if weighted:
                if slots == 2:
                    pltpu.sync_copy(ee_h.at[c, pl.ds(base, C)], wbuf)
                else:
                    pltpu.sync_copy(ee_h.at[0, pl.ds(base, C)], wbuf)

                def rbody(r, _):
                    idx16 = jnp.broadcast_to(r, (16,)).astype(jnp.int32)
                    wr = plsc.load_gather(wbuf, [idx16])
                    for k in range(D // 16):
                        sl = pl.ds(k * 16, 16)
                        rows[r, sl] = rows[r, sl] * wr
                    return 0

                lax.fori_loop(0, C, rbody, 0)
            pltpu.sync_copy(rows, acc.at[dbuf], add=True)
            return 0

        lax.fori_loop(0, nchunks, chunk, 0)
        plsc.subcore_barrier()
        pltpu.sync_copy(acc.at[pl.ds(s * RPT, RPT)],
                        out_h.at[c].at[pl.ds(s * RPT, RPT)])

    if ee is None:
        ee = jnp.zeros((slots, 8), jnp.float32)
    return kern(table, src, dst, ee)


@jax.jit
def _degree(dst):
    @functools.partial(
        pl.kernel, mesh=_mesh(),
        compiler_params=pltpu.CompilerParams(use_tc_tiling_on_sc=False, needs_layout_passes=False),
        out_type=jax.ShapeDtypeStruct((NC, NP), jnp.float32),
        scratch_types=[
            pltpu.VMEM((C,), jnp.int32),
            pltpu.VMEM((C,), jnp.float32),
            pltpu.VMEM((RPT,), jnp.float32),
            pltpu.VMEM_SHARED((NP,), jnp.float32),
            pltpu.SemaphoreType.DMA,
        ])
    def kern(dst_h, out_h, dbuf, ones, zbuf, acc, sem):
        c = lax.axis_index("c")
        s = lax.axis_index("s")
        _zero_fill(zbuf, RPT // 16)
        pltpu.sync_copy(zbuf, acc.at[pl.ds(s * RPT, RPT)])
        o = jnp.ones((16,), jnp.float32)
        for j in range(C // 16):
            ones[pl.ds(j * 16, 16)] = o
        plsc.subcore_barrier()

        def chunk(i, _):
            base = (s * NC + c) * EPW + i * C
            pltpu.sync_copy(dst_h.at[pl.ds(base, C)], dbuf)
            pltpu.sync_copy(ones, acc.at[dbuf], add=True)
            return 0

        lax.fori_loop(0, EPW // C, chunk, 0)
        plsc.subcore_barrier()
        pltpu.sync_copy(acc.at[pl.ds(s * RPT, RPT)],
                        out_h.at[c].at[pl.ds(s * RPT, RPT)])

    return kern(dst)


def _gat1_hop(h, att_s, att_d, src, dst):
    a_s = jnp.einsum('hnd,hd->hn', h, att_s)
    a_d = jnp.einsum('hnd,hd->hn', h, att_d)
    asd = jnp.stack([a_s, a_d])
    ee, den = _edge_scalars(asd, src, dst, H=HEADS)
    rden = 1.0 / (den + 1e-16)
    outs = [_spmm(h[2 * k:2 * k + 2], src, dst, ee[2 * k:2 * k + 2],
                  slots=2, weighted=True) for k in range(HEADS // 2)]
    out = jnp.concatenate(outs, axis=0)
    return out * rden[:, :, None]


def _gat2_hops(g, p, src, dst, hops):
    h = g @ p['gat2_W']
    for _ in range(hops):
        a_s = h @ p['gat2_as'][0]
        a_d = h @ p['gat2_ad'][0]
        asd = jnp.stack([a_s, a_d])[:, None, :]
        ee, den = _edge_scalars(asd, src, dst, H=1)
        rden = 1.0 / (den[0] + 1e-16)
        part = _spmm(h[None], src, dst, ee, slots=1, weighted=True)
        h = (part[0] + part[1]) * rden[:, None]
    h = h + p['gat2_b']
    h = jax.nn.relu(h)
    h = jax.nn.relu(h @ p['gatA_W'] + p['gatA_b'])
    h = jax.nn.relu(h @ p['gatB_W'] + p['gatB_b'])
    return h @ p['gatC_W'] + p['gatC_b']


def _gcn_layer(h_in, src, dst, W, b, dinv, inv_deg, hops):
    h = h_in @ W
    nslab = h.shape[1] // D
    for _ in range(hops):
        hs = h * dinv[:, None]
        slabs = hs.reshape(NP, nslab, D).transpose(1, 0, 2)
        outs = []
        for k in range(nslab // 2):
            o = _spmm(slabs[2 * k:2 * k + 2], src, dst, slots=2,
                      weighted=False)
            outs.append(o)
        agg = jnp.concatenate(outs, axis=0).transpose(1, 0, 2).reshape(NP, -1)
        h = agg * dinv[:, None] + h * inv_deg[:, None]
    return h + b


def kernel(x, edge_index, batch, params):
    p = params
    src = edge_index[0]
    dst = edge_index[1]
    xp = jnp.pad(x, ((0, NP - N), (0, 0)))

    h = (xp @ p['gat1_W']).reshape(NP, HEADS, D).transpose(1, 0, 2)
    g = []
    for _ in range(3):
        h = _gat1_hop(h, p['gat1_as'], p['gat1_ad'], src, dst)
        g.append(jax.nn.elu(
            h.transpose(1, 0, 2).reshape(NP, HEADS * D) + p['gat1_b']))

    b1 = jax.nn.relu(_gat2_hops(g[0], p, src, dst, 1))
    b2 = jax.nn.relu(_gat2_hops(g[1], p, src, dst, 2))
    h3 = jax.nn.relu(_gat2_hops(g[2], p, src, dst, 3))

    degp = _degree(dst)
    deg = 1.0 + degp[0] + degp[1]
    dinv = deg ** -0.5
    inv_deg = dinv * dinv

    h1 = jax.nn.relu(_gcn_layer(b1, src, dst, p['gcn2_W'], p['gcn2_b'],
                                dinv, inv_deg, 1))
    h1 = jax.nn.relu(_gcn_layer(h1, src, dst, p['gcn3_W'], p['gcn3_b'],
                                dinv, inv_deg, 1))
    h2 = jax.nn.relu(_gcn_layer(b2, src, dst, p['gcn2_W'], p['gcn2_b'],
                                dinv, inv_deg, 2))

    a = h1 @ p['hwA_W'] + p['hwA_b']
    b = h2 @ p['hwB_W'] + p['hwB_b']
    z = jax.nn.sigmoid(a + b)
    hmix = z * b + (1.0 - z) * a
    gi = h3 @ p['gru_Wi'] + p['gru_bi']
    gh = hmix @ p['gru_Wh'] + p['gru_bh']
    i_r, i_z, i_n = jnp.split(gi, 3, axis=-1)
    h_r, h_z, h_n = jnp.split(gh, 3, axis=-1)
    r = jax.nn.sigmoid(i_r + h_r)
    zz = jax.nn.sigmoid(i_z + h_z)
    nn_ = jnp.tanh(i_n + r * h_n)
    concat = (1.0 - zz) * nn_ + zz * hmix

    vi = jax.ops.segment_max(concat[:N], batch, num_segments=G)
    vi = jnp.where(jnp.isfinite(vi), vi, 0.0)
    vi = jax.nn.relu(vi @ p['fA_W'] + p['fA_b'])
    return vi @ p['fB_W'] + p['fB_b']


# speedup vs baseline: 20.0909x; 2.1637x over previous
import functools

import jax
import jax.numpy as jnp
from jax import lax
from jax.experimental import pallas as pl
from jax.experimental.pallas import tpu as pltpu
from jax.experimental.pallas import tpu_sc as plsc

N, E, D, G, HEADS = 10000, 320000, 128, 64, 8
NP = 10240

NC = 2
NS = 16
EPS = E // NS
EPW = E // (NS * NC)
C = 80
RPT = NP // NS

_mesh_cache = []


def _mesh():
    if not _mesh_cache:
        _mesh_cache.append(plsc.VectorSubcoreMesh(
            core_axis_name="c", subcore_axis_name="s"))
    return _mesh_cache[0]


def _zero_fill(buf, n16):
    z = jnp.zeros((16,), jnp.float32)

    def body(i, _):
        buf[pl.ds(i * 16, 16)] = z
        return 0

    lax.fori_loop(0, n16, body, 0)


def _zero_fill2d(buf, nrows, ncols):
    z = jnp.zeros((16,), jnp.float32)

    def body(r, _):
        for k in range(ncols // 16):
            buf[r, pl.ds(k * 16, 16)] = z
        return 0

    lax.fori_loop(0, nrows, body, 0)


@functools.partial(jax.jit, static_argnames=("H",))
def _edge_scalars(asd, src, dst, *, H):
    HPC = (H + 1) // 2

    @functools.partial(
        pl.kernel, mesh=_mesh(),
        compiler_params=pltpu.CompilerParams(use_tc_tiling_on_sc=False, needs_layout_passes=False),
        out_type=(jax.ShapeDtypeStruct((H, E), jnp.float32),
                  jax.ShapeDtypeStruct((H, NP), jnp.float32)),
        scratch_types=[
            pltpu.VMEM((HPC, 2, NP), jnp.float32),
            pltpu.VMEM((C,), jnp.int32),
            pltpu.VMEM((C,), jnp.int32),
            pltpu.VMEM((HPC, C), jnp.float32),
            pltpu.VMEM((RPT,), jnp.float32),
            pltpu.VMEM_SHARED((HPC, NP), jnp.float32),
            pltpu.SemaphoreType.DMA,
        ])
    def kern(asd_h, src_h, dst_h, ee_h, den_h, tabs, sbuf, dbuf, eebuf, zbuf,
             dsh, sem):
        c = lax.axis_index("c")
        s = lax.axis_index("s")
        active = c * HPC < H

        _zero_fill(zbuf, RPT // 16)
        for hh in range(HPC):
            pltpu.sync_copy(zbuf, dsh.at[hh, pl.ds(s * RPT, RPT)])
        plsc.subcore_barrier()

        @pl.when(active)
        def _():
            for hh in range(HPC):
                pltpu.sync_copy(asd_h.at[0, c * HPC + hh], tabs.at[hh, 0])
                pltpu.sync_copy(asd_h.at[1, c * HPC + hh], tabs.at[hh, 1])

            def chunk(i, _):
                base = s * EPS + i * C
                pltpu.sync_copy(src_h.at[pl.ds(base, C)], sbuf)
                pltpu.sync_copy(dst_h.at[pl.ds(base, C)], dbuf)
                for hh in range(HPC):
                    for j in range(C // 16):
                        s16 = sbuf[pl.ds(j * 16, 16)]
                        d16 = dbuf[pl.ds(j * 16, 16)]
                        av = plsc.load_gather(tabs.at[hh, 0], [s16])
                        bv = plsc.load_gather(tabs.at[hh, 1], [d16])
                        e16 = av + bv
                        e16 = jnp.where(e16 >= 0.0, e16, e16 * 0.2)
                        eebuf[hh, pl.ds(j * 16, 16)] = jnp.exp(e16)
                for hh in range(HPC):
                    pltpu.sync_copy(eebuf.at[hh],
                                    ee_h.at[c * HPC + hh, pl.ds(base, C)])
                    pltpu.sync_copy(eebuf.at[hh], dsh.at[hh].at[dbuf],
                                    add=True)
                return 0

            lax.fori_loop(0, EPS // C, chunk, 0)

        plsc.subcore_barrier()

        @pl.when(active)
        def _():
            for hh in range(HPC):
                pltpu.sync_copy(dsh.at[hh, pl.ds(s * RPT, RPT)],
                                den_h.at[c * HPC + hh, pl.ds(s * RPT, RPT)])

    return kern(asd, src, dst)


@functools.partial(jax.jit, static_argnames=("slots", "weighted"))
def _spmm(table, src2, dst2, ee2=None, *, slots, weighted):
    NCH = (EPS if slots == 2 else EPW) // C
    SEG = 50 if slots == 2 else 25
    NSEG = NCH // SEG

    @functools.partial(
        pl.kernel, mesh=_mesh(),
        compiler_params=pltpu.CompilerParams(use_tc_tiling_on_sc=False, needs_layout_passes=False),
        out_type=jax.ShapeDtypeStruct((NC, NP, D), jnp.float32),
        scratch_types=[
            pltpu.VMEM((SEG, C), jnp.int32),
            pltpu.VMEM((SEG, C), jnp.int32),
            pltpu.VMEM((SEG, C), jnp.float32),
            pltpu.VMEM((2, C, D), jnp.float32),
            pltpu.VMEM_SHARED((NP, D), jnp.float32),
            pltpu.SemaphoreType.DMA,
            pltpu.SemaphoreType.DMA,
            pltpu.SemaphoreType.DMA,
            pltpu.SemaphoreType.DMA,
        ])
    def kern(tbl_h, src_h, dst_h, ee_h, out_h, src_l, dst_l, ee_l, rows,
             acc, sg0, sg1, ss0, ss1):
        c = lax.axis_index("c")
        s = lax.axis_index("s")
        sg = (sg0, sg1)
        ss = (ss0, ss1)
        if slots == 2:
            row0 = s * NCH
        else:
            row0 = (s * NC + c) * NCH

        _zero_fill2d(rows.at[0], C, D)
        for r in range(RPT // C):
            pltpu.sync_copy(rows.at[0], acc.at[pl.ds(s * RPT + r * C, C)])
        plsc.subcore_barrier()

        tbl = tbl_h.at[c] if slots == 2 else tbl_h.at[0]
        dummy = tbl_h.at[0, pl.ds(0, C)]

        def g_issue(ci, b):
            pltpu.async_copy(tbl.at[src_l.at[ci]], rows.at[b], sg[b])

        def g_wait(b):
            pltpu.make_async_copy(dummy, rows.at[b], sg[b]).wait()

        def s_issue(ci, b):
            pltpu.async_copy(rows.at[b], acc.at[dst_l.at[ci]], ss[b],
                             add=True)

        def s_wait(b):
            pltpu.make_async_copy(dummy, rows.at[b], ss[b]).wait()

        def scale(ci, b):
            if not weighted:
                return
            rb = rows.at[b]

            def jbody(jj, _):
                w16 = ee_l[ci, pl.ds(jj * 16, 16)]
                for l in range(16):
                    wr = w16[l]
                    for k in range(D // 16):
                        sl = pl.ds(k * 16, 16)
                        rb[jj * 16 + l, sl] = rb[jj * 16 + l, sl] * wr
                return 0

            lax.fori_loop(0, C // 16, jbody, 0)

        def seg_body(g, _):
            pltpu.sync_copy(src_h.at[pl.ds(row0 + g * SEG, SEG)], src_l)
            pltpu.sync_copy(dst_h.at[pl.ds(row0 + g * SEG, SEG)], dst_l)
            if weighted:
                if slots == 2:
                    pltpu.sync_copy(ee_h.at[c].at[pl.ds(row0 + g * SEG, SEG)],
                                    ee_l)
                else:
                    pltpu.sync_copy(ee_h.at[0].at[pl.ds(row0 + g * SEG, SEG)],
                                    ee_l)
            g_issue(0, 0)

            def pair(i, _):
                c0 = 2 * i
                c1 = c0 + 1

                @pl.when(i > 0)
                def _():
                    s_wait(1)
                g_issue(c1, 1)
                g_wait(0)
                scale(c0, 0)
                s_issue(c0, 0)
                g_wait(1)
                scale(c1, 1)
                s_wait(0)
                @pl.when(c0 + 2 < SEG)
                def _():
                    g_issue(c0 + 2, 0)
                s_issue(c1, 1)
                return 0

            lax.fori_loop(0, SEG // 2, pair, 0)
            if SEG % 2 == 1:
                g_wait(0)
                scale(SEG - 1, 0)
                s_issue(SEG - 1, 0)
                s_wait(0)
            s_wait(1)
            return 0

        lax.fori_loop(0, NSEG, seg_body, 0)

        plsc.subcore_barrier()
        pltpu.sync_copy(acc.at[pl.ds(s * RPT, RPT)],
                        out_h.at[c].at[pl.ds(s * RPT, RPT)])

    if ee2 is None:
        ee2 = jnp.zeros((slots, NCH, C), jnp.float32)
    return kern(table, src2, dst2, ee2)


@jax.jit
def _degree(dst):
    @functools.partial(
        pl.kernel, mesh=_mesh(),
        compiler_params=pltpu.CompilerParams(use_tc_tiling_on_sc=False, needs_layout_passes=False),
        out_type=jax.ShapeDtypeStruct((NC, NP), jnp.float32),
        scratch_types=[
            pltpu.VMEM((C,), jnp.int32),
            pltpu.VMEM((C,), jnp.float32),
            pltpu.VMEM((RPT,), jnp.float32),
            pltpu.VMEM_SHARED((NP,), jnp.float32),
            pltpu.SemaphoreType.DMA,
        ])
    def kern(dst_h, out_h, dbuf, ones, zbuf, acc, sem):
        c = lax.axis_index("c")
        s = lax.axis_index("s")
        _zero_fill(zbuf, RPT // 16)
        pltpu.sync_copy(zbuf, acc.at[pl.ds(s * RPT, RPT)])
        o = jnp.ones((16,), jnp.float32)
        for j in range(C // 16):
            ones[pl.ds(j * 16, 16)] = o
        plsc.subcore_barrier()

        def chunk(i, _):
            base = (s * NC + c) * EPW + i * C
            pltpu.sync_copy(dst_h.at[pl.ds(base, C)], dbuf)
            pltpu.sync_copy(ones, acc.at[dbuf], add=True)
            return 0

        lax.fori_loop(0, EPW // C, chunk, 0)
        plsc.subcore_barrier()
        pltpu.sync_copy(acc.at[pl.ds(s * RPT, RPT)],
                        out_h.at[c].at[pl.ds(s * RPT, RPT)])

    return kern(dst)


def _gat1_hop(h, att_s, att_d, src, dst, src2, dst2):
    a_s = jnp.einsum('hnd,hd->hn', h, att_s)
    a_d = jnp.einsum('hnd,hd->hn', h, att_d)
    asd = jnp.stack([a_s, a_d])
    ee, den = _edge_scalars(asd, src, dst, H=HEADS)
    rden = 1.0 / (den + 1e-16)
    ee2 = ee.reshape(HEADS, E // C, C)
    outs = [_spmm(h[2 * k:2 * k + 2], src2, dst2, ee2[2 * k:2 * k + 2],
                  slots=2, weighted=True) for k in range(HEADS // 2)]
    out = jnp.concatenate(outs, axis=0)
    return out * rden[:, :, None]


def _gat2_hops(g, p, src, dst, src2, dst2, hops):
    h = g @ p['gat2_W']
    for _ in range(hops):
        a_s = h @ p['gat2_as'][0]
        a_d = h @ p['gat2_ad'][0]
        asd = jnp.stack([a_s, a_d])[:, None, :]
        ee, den = _edge_scalars(asd, src, dst, H=1)
        rden = 1.0 / (den[0] + 1e-16)
        part = _spmm(h[None], src2, dst2, ee.reshape(1, E // C, C),
                     slots=1, weighted=True)
        h = (part[0] + part[1]) * rden[:, None]
    h = h + p['gat2_b']
    h = jax.nn.relu(h)
    h = jax.nn.relu(h @ p['gatA_W'] + p['gatA_b'])
    h = jax.nn.relu(h @ p['gatB_W'] + p['gatB_b'])
    return h @ p['gatC_W'] + p['gatC_b']


def _gcn_layer(h_in, src2, dst2, W, b, dinv, inv_deg, hops):
    h = h_in @ W
    nslab = h.shape[1] // D
    for _ in range(hops):
        hs = h * dinv[:, None]
        slabs = hs.reshape(NP, nslab, D).transpose(1, 0, 2)
        outs = []
        for k in range(nslab // 2):
            o = _spmm(slabs[2 * k:2 * k + 2], src2, dst2, slots=2,
                      weighted=False)
            outs.append(o)
        agg = jnp.concatenate(outs, axis=0).transpose(1, 0, 2).reshape(NP, -1)
        h = agg * dinv[:, None] + h * inv_deg[:, None]
    return h + b


def kernel(x, edge_index, batch, params):
    p = params
    src = edge_index[0]
    dst = edge_index[1]
    src2 = src.reshape(E // C, C)
    dst2 = dst.reshape(E // C, C)
    xp = jnp.pad(x, ((0, NP - N), (0, 0)))

    h = (xp @ p['gat1_W']).reshape(NP, HEADS, D).transpose(1, 0, 2)
    g = []
    for _ in range(3):
        h = _gat1_hop(h, p['gat1_as'], p['gat1_ad'], src, dst, src2, dst2)
        g.append(jax.nn.elu(
            h.transpose(1, 0, 2).reshape(NP, HEADS * D) + p['gat1_b']))

    b1 = jax.nn.relu(_gat2_hops(g[0], p, src, dst, src2, dst2, 1))
    b2 = jax.nn.relu(_gat2_hops(g[1], p, src, dst, src2, dst2, 2))
    h3 = jax.nn.relu(_gat2_hops(g[2], p, src, dst, src2, dst2, 3))

    degp = _degree(dst)
    deg = 1.0 + degp[0] + degp[1]
    dinv = deg ** -0.5
    inv_deg = dinv * dinv

    h1 = jax.nn.relu(_gcn_layer(b1, src2, dst2, p['gcn2_W'], p['gcn2_b'],
                                dinv, inv_deg, 1))
    h1 = jax.nn.relu(_gcn_layer(h1, src2, dst2, p['gcn3_W'], p['gcn3_b'],
                                dinv, inv_deg, 1))
    h2 = jax.nn.relu(_gcn_layer(b2, src2, dst2, p['gcn2_W'], p['gcn2_b'],
                                dinv, inv_deg, 2))

    a = h1 @ p['hwA_W'] + p['hwA_b']
    b = h2 @ p['hwB_W'] + p['hwB_b']
    z = jax.nn.sigmoid(a + b)
    hmix = z * b + (1.0 - z) * a
    gi = h3 @ p['gru_Wi'] + p['gru_bi']
    gh = hmix @ p['gru_Wh'] + p['gru_bh']
    i_r, i_z, i_n = jnp.split(gi, 3, axis=-1)
    h_r, h_z, h_n = jnp.split(gh, 3, axis=-1)
    r = jax.nn.sigmoid(i_r + h_r)
    zz = jax.nn.sigmoid(i_z + h_z)
    nn_ = jnp.tanh(i_n + r * h_n)
    concat = (1.0 - zz) * nn_ + zz * hmix

    vi = jax.ops.segment_max(concat[:N], batch, num_segments=G)
    vi = jnp.where(jnp.isfinite(vi), vi, 0.0)
    vi = jax.nn.relu(vi @ p['fA_W'] + p['fA_b'])
    return vi @ p['fB_W'] + p['fB_b']


# speedup vs baseline: 27.1776x; 1.3527x over previous
import functools

import jax
import jax.numpy as jnp
from jax import lax
from jax.experimental import pallas as pl
from jax.experimental.pallas import tpu as pltpu
from jax.experimental.pallas import tpu_sc as plsc

N, E, D, G, HEADS = 10000, 320000, 128, 64, 8
NP = 10240

NC = 2
NS = 16
EPS = E // NS
EPW = E // (NS * NC)
C = 80
RPT = NP // NS

_mesh_cache = []


def _mesh():
    if not _mesh_cache:
        _mesh_cache.append(plsc.VectorSubcoreMesh(
            core_axis_name="c", subcore_axis_name="s"))
    return _mesh_cache[0]


def _zero_fill(buf, n16):
    z = jnp.zeros((16,), jnp.float32)

    def body(i, _):
        buf[pl.ds(i * 16, 16)] = z
        return 0

    lax.fori_loop(0, n16, body, 0)


def _zero_fill2d(buf, nrows, ncols):
    z = jnp.zeros((16,), jnp.float32)

    def body(r, _):
        for k in range(ncols // 16):
            buf[r, pl.ds(k * 16, 16)] = z
        return 0

    lax.fori_loop(0, nrows, body, 0)


@functools.partial(jax.jit, static_argnames=("H",))
def _edge_scalars(asd, src2, dst2, *, H):
    HPC = max(H // NC, 1)
    SEG = 125
    NCH = EPS // C if H > 1 else EPW // C
    NSEG = NCH // SEG

    @functools.partial(
        pl.kernel, mesh=_mesh(),
        compiler_params=pltpu.CompilerParams(use_tc_tiling_on_sc=False, needs_layout_passes=False),
        out_type=(jax.ShapeDtypeStruct((H, E), jnp.float32),
                  jax.ShapeDtypeStruct((NC * HPC, NP), jnp.float32)),
        scratch_types=[
            pltpu.VMEM((HPC, 2, NP), jnp.float32),
            pltpu.VMEM((SEG, C), jnp.int32),
            pltpu.VMEM((SEG, C), jnp.int32),
            pltpu.VMEM((2, HPC, C), jnp.float32),
            pltpu.VMEM((RPT,), jnp.float32),
            pltpu.VMEM_SHARED((HPC, NP), jnp.float32),
            pltpu.SemaphoreType.DMA,
            pltpu.SemaphoreType.DMA,
            pltpu.SemaphoreType.DMA,
            pltpu.SemaphoreType.DMA,
        ])
    def kern(asd_h, src_h, dst_h, ee_h, den_h, tabs, src_l, dst_l, eebuf,
             zbuf, dsh, st0, st1, sc0, sc1):
        c = lax.axis_index("c")
        s = lax.axis_index("s")
        sst = (st0, st1)
        ssc = (sc0, sc1)
        if H > 1:
            row0 = s * NCH
        else:
            row0 = (s * NC + c) * NCH

        _zero_fill(zbuf, RPT // 16)
        for hh in range(HPC):
            pltpu.sync_copy(zbuf, dsh.at[hh, pl.ds(s * RPT, RPT)])
        plsc.subcore_barrier()

        for hh in range(HPC):
            if H > 1:
                pltpu.sync_copy(asd_h.at[0, c * HPC + hh], tabs.at[hh, 0])
                pltpu.sync_copy(asd_h.at[1, c * HPC + hh], tabs.at[hh, 1])
            else:
                pltpu.sync_copy(asd_h.at[0, 0], tabs.at[hh, 0])
                pltpu.sync_copy(asd_h.at[1, 0], tabs.at[hh, 1])

        dummy = ee_h.at[0, pl.ds(0, C)]

        def compute(ci, b):
            for hh in range(HPC):
                for j in range(C // 16):
                    s16 = src_l[ci, pl.ds(j * 16, 16)]
                    d16 = dst_l[ci, pl.ds(j * 16, 16)]
                    av = plsc.load_gather(tabs.at[hh, 0], [s16])
                    bv = plsc.load_gather(tabs.at[hh, 1], [d16])
                    e16 = av + bv
                    e16 = jnp.where(e16 >= 0.0, e16, e16 * 0.2)
                    eebuf[b, hh, pl.ds(j * 16, 16)] = jnp.exp(e16)

        def issue(g, ci, b):
            gbase = (row0 + g * SEG + ci) * C
            for hh in range(HPC):
                hrow = (c * HPC + hh) if H > 1 else 0
                pltpu.async_copy(eebuf.at[b, hh],
                                 ee_h.at[hrow, pl.ds(gbase, C)], sst[b])
                pltpu.async_copy(eebuf.at[b, hh], dsh.at[hh].at[dst_l.at[ci]],
                                 ssc[b], add=True)

        def drain(b):
            for hh in range(HPC):
                pltpu.make_async_copy(dummy, eebuf.at[b, hh], sst[b]).wait()
                pltpu.make_async_copy(dummy, eebuf.at[b, hh], ssc[b]).wait()

        def seg_body(g, _):
            pltpu.sync_copy(src_h.at[pl.ds(row0 + g * SEG, SEG)], src_l)
            pltpu.sync_copy(dst_h.at[pl.ds(row0 + g * SEG, SEG)], dst_l)

            def pair(i, _):
                c0 = 2 * i
                c1 = c0 + 1

                @pl.when(i > 0)
                def _():
                    drain(0)
                compute(c0, 0)
                issue(g, c0, 0)

                @pl.when(i > 0)
                def _():
                    drain(1)
                compute(c1, 1)
                issue(g, c1, 1)
                return 0

            lax.fori_loop(0, SEG // 2, pair, 0)
            if SEG % 2 == 1:
                drain(0)
                compute(SEG - 1, 0)
                issue(g, SEG - 1, 0)
            drain(0)
            drain(1)
            return 0

        lax.fori_loop(0, NSEG, seg_body, 0)
        plsc.subcore_barrier()

        for hh in range(HPC):
            pltpu.sync_copy(dsh.at[hh, pl.ds(s * RPT, RPT)],
                            den_h.at[c * HPC + hh, pl.ds(s * RPT, RPT)])

    return kern(asd, src2, dst2)


@functools.partial(jax.jit, static_argnames=("slots", "weighted"))
def _spmm(table, src2, dst2, ee2=None, *, slots, weighted):
    NCH = (EPS if slots == 2 else EPW) // C
    SEG = 50 if slots == 2 else 25
    NSEG = NCH // SEG

    @functools.partial(
        pl.kernel, mesh=_mesh(),
        compiler_params=pltpu.CompilerParams(use_tc_tiling_on_sc=False, needs_layout_passes=False),
        out_type=jax.ShapeDtypeStruct((NC, NP, D), jnp.float32),
        scratch_types=[
            pltpu.VMEM((SEG, C), jnp.int32),
            pltpu.VMEM((SEG, C), jnp.int32),
            pltpu.VMEM((SEG, C), jnp.float32),
            pltpu.VMEM((2, C, D), jnp.float32),
            pltpu.VMEM_SHARED((NP, D), jnp.float32),
            pltpu.SemaphoreType.DMA,
            pltpu.SemaphoreType.DMA,
            pltpu.SemaphoreType.DMA,
            pltpu.SemaphoreType.DMA,
        ])
    def kern(tbl_h, src_h, dst_h, ee_h, out_h, src_l, dst_l, ee_l, rows,
             acc, sg0, sg1, ss0, ss1):
        c = lax.axis_index("c")
        s = lax.axis_index("s")
        sg = (sg0, sg1)
        ss = (ss0, ss1)
        if slots == 2:
            row0 = s * NCH
        else:
            row0 = (s * NC + c) * NCH

        _zero_fill2d(rows.at[0], C, D)
        for r in range(RPT // C):
            pltpu.sync_copy(rows.at[0], acc.at[pl.ds(s * RPT + r * C, C)])
        plsc.subcore_barrier()

        tbl = tbl_h.at[c] if slots == 2 else tbl_h.at[0]
        dummy = tbl_h.at[0, pl.ds(0, C)]

        def g_issue(ci, b):
            pltpu.async_copy(tbl.at[src_l.at[ci]], rows.at[b], sg[b])

        def g_wait(b):
            pltpu.make_async_copy(dummy, rows.at[b], sg[b]).wait()

        def s_issue(ci, b):
            pltpu.async_copy(rows.at[b], acc.at[dst_l.at[ci]], ss[b],
                             add=True)

        def s_wait(b):
            pltpu.make_async_copy(dummy, rows.at[b], ss[b]).wait()

        def scale(ci, b):
            if not weighted:
                return
            rb = rows.at[b]

            def jbody(jj, _):
                w16 = ee_l[ci, pl.ds(jj * 16, 16)]
                for l in range(16):
                    wr = w16[l]
                    for k in range(D // 16):
                        sl = pl.ds(k * 16, 16)
                        rb[jj * 16 + l, sl] = rb[jj * 16 + l, sl] * wr
                return 0

            lax.fori_loop(0, C // 16, jbody, 0)

        def seg_body(g, _):
            pltpu.sync_copy(src_h.at[pl.ds(row0 + g * SEG, SEG)], src_l)
            pltpu.sync_copy(dst_h.at[pl.ds(row0 + g * SEG, SEG)], dst_l)
            if weighted:
                if slots == 2:
                    pltpu.sync_copy(ee_h.at[c].at[pl.ds(row0 + g * SEG, SEG)],
                                    ee_l)
                else:
                    pltpu.sync_copy(ee_h.at[0].at[pl.ds(row0 + g * SEG, SEG)],
                                    ee_l)
            g_issue(0, 0)

            def pair(i, _):
                c0 = 2 * i
                c1 = c0 + 1

                @pl.when(i > 0)
                def _():
                    s_wait(1)
                g_issue(c1, 1)
                g_wait(0)
                scale(c0, 0)
                s_issue(c0, 0)
                g_wait(1)
                scale(c1, 1)
                s_wait(0)
                @pl.when(c0 + 2 < SEG)
                def _():
                    g_issue(c0 + 2, 0)
                s_issue(c1, 1)
                return 0

            lax.fori_loop(0, SEG // 2, pair, 0)
            if SEG % 2 == 1:
                g_wait(0)
                scale(SEG - 1, 0)
                s_issue(SEG - 1, 0)
                s_wait(0)
            s_wait(1)
            return 0

        lax.fori_loop(0, NSEG, seg_body, 0)

        plsc.subcore_barrier()
        pltpu.sync_copy(acc.at[pl.ds(s * RPT, RPT)],
                        out_h.at[c].at[pl.ds(s * RPT, RPT)])

    if ee2 is None:
        ee2 = jnp.zeros((slots, NCH, C), jnp.float32)
    return kern(table, src2, dst2, ee2)


@jax.jit
def _degree(dst):
    @functools.partial(
        pl.kernel, mesh=_mesh(),
        compiler_params=pltpu.CompilerParams(use_tc_tiling_on_sc=False, needs_layout_passes=False),
        out_type=jax.ShapeDtypeStruct((NC, NP), jnp.float32),
        scratch_types=[
            pltpu.VMEM((C,), jnp.int32),
            pltpu.VMEM((C,), jnp.float32),
            pltpu.VMEM((RPT,), jnp.float32),
            pltpu.VMEM_SHARED((NP,), jnp.float32),
            pltpu.SemaphoreType.DMA,
        ])
    def kern(dst_h, out_h, dbuf, ones, zbuf, acc, sem):
        c = lax.axis_index("c")
        s = lax.axis_index("s")
        _zero_fill(zbuf, RPT // 16)
        pltpu.sync_copy(zbuf, acc.at[pl.ds(s * RPT, RPT)])
        o = jnp.ones((16,), jnp.float32)
        for j in range(C // 16):
            ones[pl.ds(j * 16, 16)] = o
        plsc.subcore_barrier()

        def chunk(i, _):
            base = (s * NC + c) * EPW + i * C
            pltpu.sync_copy(dst_h.at[pl.ds(base, C)], dbuf)
            pltpu.sync_copy(ones, acc.at[dbuf], add=True)
            return 0

        lax.fori_loop(0, EPW // C, chunk, 0)
        plsc.subcore_barrier()
        pltpu.sync_copy(acc.at[pl.ds(s * RPT, RPT)],
                        out_h.at[c].at[pl.ds(s * RPT, RPT)])

    return kern(dst)


def _gat1_hop(h, att_s, att_d, src, dst, src2, dst2):
    a_s = jnp.einsum('hnd,hd->hn', h, att_s)
    a_d = jnp.einsum('hnd,hd->hn', h, att_d)
    asd = jnp.stack([a_s, a_d])
    ee, den = _edge_scalars(asd, src2, dst2, H=HEADS)
    rden = 1.0 / (den + 1e-16)
    ee2 = ee.reshape(HEADS, E // C, C)
    outs = [_spmm(h[2 * k:2 * k + 2], src2, dst2, ee2[2 * k:2 * k + 2],
                  slots=2, weighted=True) for k in range(HEADS // 2)]
    out = jnp.concatenate(outs, axis=0)
    return out * rden[:, :, None]


def _gat2_hops(g, p, src, dst, src2, dst2, hops):
    h = g @ p['gat2_W']
    for _ in range(hops):
        a_s = h @ p['gat2_as'][0]
        a_d = h @ p['gat2_ad'][0]
        asd = jnp.stack([a_s, a_d])[:, None, :]
        ee, den = _edge_scalars(asd, src2, dst2, H=1)
        rden = 1.0 / (den[0] + den[1] + 1e-16)
        part = _spmm(h[None], src2, dst2, ee.reshape(1, E // C, C),
                     slots=1, weighted=True)
        h = (part[0] + part[1]) * rden[:, None]
    h = h + p['gat2_b']
    h = jax.nn.relu(h)
    h = jax.nn.relu(h @ p['gatA_W'] + p['gatA_b'])
    h = jax.nn.relu(h @ p['gatB_W'] + p['gatB_b'])
    return h @ p['gatC_W'] + p['gatC_b']


def _gcn_layer(h_in, src2, dst2, W, b, dinv, inv_deg, hops):
    h = h_in @ W
    nslab = h.shape[1] // D
    for _ in range(hops):
        hs = h * dinv[:, None]
        slabs = hs.reshape(NP, nslab, D).transpose(1, 0, 2)
        outs = []
        for k in range(nslab // 2):
            o = _spmm(slabs[2 * k:2 * k + 2], src2, dst2, slots=2,
                      weighted=False)
            outs.append(o)
        agg = jnp.concatenate(outs, axis=0).transpose(1, 0, 2).reshape(NP, -1)
        h = agg * dinv[:, None] + h * inv_deg[:, None]
    return h + b


def kernel(x, edge_index, batch, params):
    p = params
    src = edge_index[0]
    dst = edge_index[1]
    src2 = src.reshape(E // C, C)
    dst2 = dst.reshape(E // C, C)
    xp = jnp.pad(x, ((0, NP - N), (0, 0)))

    h = (xp @ p['gat1_W']).reshape(NP, HEADS, D).transpose(1, 0, 2)
    g = []
    for _ in range(3):
        h = _gat1_hop(h, p['gat1_as'], p['gat1_ad'], src, dst, src2, dst2)
        g.append(jax.nn.elu(
            h.transpose(1, 0, 2).reshape(NP, HEADS * D) + p['gat1_b']))

    b1 = jax.nn.relu(_gat2_hops(g[0], p, src, dst, src2, dst2, 1))
    b2 = jax.nn.relu(_gat2_hops(g[1], p, src, dst, src2, dst2, 2))
    h3 = jax.nn.relu(_gat2_hops(g[2], p, src, dst, src2, dst2, 3))

    degp = _degree(dst)
    deg = 1.0 + degp[0] + degp[1]
    dinv = deg ** -0.5
    inv_deg = dinv * dinv

    h1 = jax.nn.relu(_gcn_layer(b1, src2, dst2, p['gcn2_W'], p['gcn2_b'],
                                dinv, inv_deg, 1))
    h1 = jax.nn.relu(_gcn_layer(h1, src2, dst2, p['gcn3_W'], p['gcn3_b'],
                                dinv, inv_deg, 1))
    h2 = jax.nn.relu(_gcn_layer(b2, src2, dst2, p['gcn2_W'], p['gcn2_b'],
                                dinv, inv_deg, 2))

    a = h1 @ p['hwA_W'] + p['hwA_b']
    b = h2 @ p['hwB_W'] + p['hwB_b']
    z = jax.nn.sigmoid(a + b)
    hmix = z * b + (1.0 - z) * a
    gi = h3 @ p['gru_Wi'] + p['gru_bi']
    gh = hmix @ p['gru_Wh'] + p['gru_bh']
    i_r, i_z, i_n = jnp.split(gi, 3, axis=-1)
    h_r, h_z, h_n = jnp.split(gh, 3, axis=-1)
    r = jax.nn.sigmoid(i_r + h_r)
    zz = jax.nn.sigmoid(i_z + h_z)
    nn_ = jnp.tanh(i_n + r * h_n)
    concat = (1.0 - zz) * nn_ + zz * hmix

    vi = jax.ops.segment_max(concat[:N], batch, num_segments=G)
    vi = jnp.where(jnp.isfinite(vi), vi, 0.0)
    vi = jax.nn.relu(vi @ p['fA_W'] + p['fA_b'])
    return vi @ p['fB_W'] + p['fB_b']
